# baseline (speedup 1.0000x reference)
import functools

import jax
import jax.numpy as jnp
import numpy as np
from jax import lax
from jax.experimental import pallas as pl
from jax.experimental.pallas import tpu as pltpu
from jax.experimental.pallas import tpu_sc as plsc

D_MODEL = 1024
MAX_LEN = 8192
NC = 2
NS = 16
NW = NC * NS
LANES = 16
NBUF = 3


def _pos_encoding(max_len, d_model):
    pos = np.arange(max_len, dtype=np.float32)[:, None]
    _2i = np.arange(0, d_model, 2, dtype=np.float32)
    enc = np.zeros((max_len, d_model), dtype=np.float32)
    angle = (pos / np.float_power(10000.0, (_2i / d_model))).astype(np.float32)
    enc[:, 0::2] = np.sin(angle)
    enc[:, 1::2] = np.cos(angle)
    return enc


_PE = _pos_encoding(MAX_LEN, D_MODEL)


def _make_sc_kernel(bsz, seq, D, C):
    ppw = seq // NW
    nchunk = ppw // C
    rows_c = bsz * C
    groups = D // LANES
    GB = 16
    gblocks = groups // GB
    mesh = plsc.VectorSubcoreMesh(core_axis_name="c", subcore_axis_name="s")

    @functools.partial(
        pl.kernel,
        out_type=jax.ShapeDtypeStruct((bsz * seq, D), jnp.float32),
        mesh=mesh,
        scratch_types=(
            [pltpu.VMEM((bsz, ppw), jnp.int32)]
            + [pltpu.VMEM((rows_c, D), jnp.float32)] * NBUF
            + [pltpu.VMEM((C, D), jnp.float32)] * NBUF
            + [pltpu.SemaphoreType.DMA] * (3 * NBUF)
        ),
    )
    def k(x_hbm, tok_hbm, pe_hbm, out_hbm, idx_v, *bufs):
        rows = bufs[0:NBUF]
        pes = bufs[NBUF:2 * NBUF]
        sgs = bufs[2 * NBUF:3 * NBUF]
        sps = bufs[3 * NBUF:4 * NBUF]
        ws = bufs[4 * NBUF:5 * NBUF]

        wid = lax.axis_index("s") * NC + lax.axis_index("c")
        pbase = wid * ppw
        for b in range(bsz):
            pltpu.async_copy(x_hbm.at[b, pl.ds(pbase, ppw)],
                             idx_v.at[b], sps[0])
        pltpu.make_async_copy(
            x_hbm.at[pl.ds(0, bsz), pl.ds(0, ppw)], idx_v, sps[0]).wait()

        def issue(ci, q):
            for b in range(bsz):
                pltpu.async_copy(
                    tok_hbm.at[idx_v.at[b, pl.ds(ci * C, C)]],
                    rows[q].at[pl.ds(b * C, C)], sgs[q])
            pltpu.async_copy(pe_hbm.at[pl.ds(pbase + ci * C, C)],
                             pes[q], sps[q])

        def phase(ci, q):
            qn = (q + 1) % NBUF
            @pl.when(ci >= NBUF - 1)
            def _():
                pltpu.make_async_copy(
                    tok_hbm.at[pl.ds(0, rows_c)], rows[qn], ws[qn]).wait()

            @pl.when(ci + 1 < nchunk)
            def _():
                issue(ci + 1, qn)

            pltpu.make_async_copy(
                tok_hbm.at[pl.ds(0, rows_c)], rows[q], sgs[q]).wait()
            pltpu.make_async_copy(
                pe_hbm.at[pl.ds(0, C)], pes[q], sps[q]).wait()

            for b in range(bsz):
                def add(r, carry, b=b):
                    rr = b * C + r
                    for g in range(groups):
                        sl = pl.ds(g * LANES, LANES)
                        rows[q][rr, sl] = rows[q][rr, sl] + pes[q][r, sl]
                    return carry

                lax.fori_loop(0, C, add, 0)
                pltpu.async_copy(
                    rows[q].at[pl.ds(b * C, C)],
                    out_hbm.at[pl.ds(b * seq + pbase + ci * C, C)],
                    ws[q])

        issue(0, 0)

        def body(it, carry):
            for p in range(NBUF):
                phase(it * NBUF + p, p)
            return carry

        lax.fori_loop(0, nchunk // NBUF, body, 0)
        for ci in range(nchunk - nchunk % NBUF, nchunk):
            phase(ci, ci % NBUF)

        for ci in range(nchunk - NBUF + 1, nchunk):
            pltpu.make_async_copy(
                tok_hbm.at[pl.ds(0, rows_c)], rows[ci % NBUF],
                ws[ci % NBUF]).wait()

    return k


def kernel(x, tok_emb):
    bsz, seq = x.shape
    D = tok_emb.shape[1]
    C = 8
    pe = jnp.asarray(_PE[:seq, :])
    out = _make_sc_kernel(bsz, seq, D, C)(
        x.astype(jnp.int32), tok_emb, pe)
    return out.reshape(bsz, seq, D)


# speedup vs baseline: 1.1166x; 1.1166x over previous
import functools

import jax
import jax.numpy as jnp
import numpy as np
from jax import lax
from jax.experimental import pallas as pl
from jax.experimental.pallas import tpu as pltpu
from jax.experimental.pallas import tpu_sc as plsc

D_MODEL = 1024
MAX_LEN = 8192
NC = 2
NS = 16
NW = NC * NS
LANES = 16
NBUF = 3


def _pos_encoding(max_len, d_model):
    pos = np.arange(max_len, dtype=np.float32)[:, None]
    _2i = np.arange(0, d_model, 2, dtype=np.float32)
    enc = np.zeros((max_len, d_model), dtype=np.float32)
    angle = (pos / np.float_power(10000.0, (_2i / d_model))).astype(np.float32)
    enc[:, 0::2] = np.sin(angle)
    enc[:, 1::2] = np.cos(angle)
    return enc


_PE = _pos_encoding(MAX_LEN, D_MODEL)


def _make_sc_kernel(bsz, seq, D, C):
    ppw = seq // NW
    nchunk = ppw // C
    rows_c = bsz * C
    groups = D // LANES
    GB = 16
    gblocks = groups // GB
    mesh = plsc.VectorSubcoreMesh(core_axis_name="c", subcore_axis_name="s")

    @functools.partial(
        pl.kernel,
        out_type=jax.ShapeDtypeStruct((bsz * seq, D), jnp.float32),
        mesh=mesh,
        scratch_types=(
            [pltpu.VMEM((bsz, ppw), jnp.int32)]
            + [pltpu.VMEM((rows_c, D), jnp.float32)] * NBUF
            + [pltpu.VMEM((C, D), jnp.float32)] * NBUF
            + [pltpu.SemaphoreType.DMA] * (3 * NBUF)
        ),
    )
    def k(x_hbm, tok_hbm, pe_hbm, out_hbm, idx_v, *bufs):
        rows = bufs[0:NBUF]
        pes = bufs[NBUF:2 * NBUF]
        sgs = bufs[2 * NBUF:3 * NBUF]
        sps = bufs[3 * NBUF:4 * NBUF]
        ws = bufs[4 * NBUF:5 * NBUF]

        wid = lax.axis_index("s") * NC + lax.axis_index("c")
        pbase = wid * ppw
        for b in range(bsz):
            pltpu.async_copy(x_hbm.at[b, pl.ds(pbase, ppw)],
                             idx_v.at[b], sps[0])
        pltpu.make_async_copy(
            x_hbm.at[pl.ds(0, bsz), pl.ds(0, ppw)], idx_v, sps[0]).wait()

        def issue(ci, q):
            for b in range(bsz):
                pltpu.async_copy(
                    tok_hbm.at[idx_v.at[b, pl.ds(ci * C, C)]],
                    rows[q].at[pl.ds(b * C, C)], sgs[q])
            pltpu.async_copy(pe_hbm.at[pl.ds(pbase + ci * C, C)],
                             pes[q], sps[q])

        def phase(ci, q):
            qn = (q + 1) % NBUF
            @pl.when(ci >= NBUF - 1)
            def _():
                pltpu.make_async_copy(
                    tok_hbm.at[pl.ds(0, rows_c)], rows[qn], ws[qn]).wait()

            @pl.when(ci + 1 < nchunk)
            def _():
                issue(ci + 1, qn)

            pltpu.make_async_copy(
                tok_hbm.at[pl.ds(0, rows_c)], rows[q], sgs[q]).wait()
            pltpu.make_async_copy(
                pe_hbm.at[pl.ds(0, C)], pes[q], sps[q]).wait()

            def add(t, carry):
                r = t // gblocks
                goff = (t % gblocks) * (GB * LANES)
                pv = [pes[q][r, pl.ds(goff + g * LANES, LANES)]
                      for g in range(GB)]
                for b in range(bsz):
                    rr = b * C + r
                    for g in range(GB):
                        sl = pl.ds(goff + g * LANES, LANES)
                        rows[q][rr, sl] = rows[q][rr, sl] + pv[g]
                return carry

            lax.fori_loop(0, C * gblocks, add, 0)

            for b in range(bsz):
                pltpu.async_copy(
                    rows[q].at[pl.ds(b * C, C)],
                    out_hbm.at[pl.ds(b * seq + pbase + ci * C, C)],
                    ws[q])

        issue(0, 0)

        def body(it, carry):
            for p in range(NBUF):
                phase(it * NBUF + p, p)
            return carry

        lax.fori_loop(0, nchunk // NBUF, body, 0)
        for ci in range(nchunk - nchunk % NBUF, nchunk):
            phase(ci, ci % NBUF)

        for ci in range(nchunk - NBUF + 1, nchunk):
            pltpu.make_async_copy(
                tok_hbm.at[pl.ds(0, rows_c)], rows[ci % NBUF],
                ws[ci % NBUF]).wait()

    return k


def kernel(x, tok_emb):
    bsz, seq = x.shape
    D = tok_emb.shape[1]
    C = 8
    pe = jnp.asarray(_PE[:seq, :])
    out = _make_sc_kernel(bsz, seq, D, C)(
        x.astype(jnp.int32), tok_emb, pe)
    return out.reshape(bsz, seq, D)
